# baseline (speedup 1.0000x reference)
import functools
import jax
import jax.numpy as jnp
from jax.experimental import pallas as pl


def _body(d_ref, r_ref, e_ref, out_ref):
    D = d_ref[...]
    R = r_ref[...]
    E = e_ref[...]
    sub = jax.lax.broadcasted_iota(jnp.int32, (128, 1), 0)
    lane = jax.lax.broadcasted_iota(jnp.int32, (1, 128), 1)
    I = lane * 128 + sub
    IE = I.astype(jnp.float32) * 2.0 + E
    gamma = jnp.max(R)

    bits = [((I >> n) & 1) for n in range(14)]
    zero_bits = jnp.zeros((128, 128), jnp.int32)

    def pmat(t):
        return ((sub ^ t) == lane).astype(jnp.float32)

    def lane_partner(X, t):
        return jnp.dot(X, pmat(t), preferred_element_type=jnp.float32,
                       precision=jax.lax.Precision.HIGHEST)

    def sub_partner(X, s):
        G = 128 // (2 * s)
        X4 = X.reshape(G, 2, s, 128)
        X4s = jnp.concatenate([X4[:, 1:2], X4[:, 0:1]], axis=1)
        return X4s.reshape(128, 128)

    for m in range(1, 15):
        bk = bits[m] if m < 14 else zero_bits
        for j_exp in range(m - 1, -1, -1):
            bj = bits[j_exp]
            keep = bk == bj
            if j_exp <= 6:
                s = 1 << j_exp
                Dq = sub_partner(D, s)
                IEq = sub_partner(IE, s)
                Rq = sub_partner(R, s)
            else:
                t = 1 << (j_exp - 7)
                Dq = lane_partner(D, t)
                IEq = lane_partner(IE, t)
                Rq = lane_partner(R, t)
            pre = (D > Dq) | ((D == Dq) & (IE < IEq))
            take = pre == keep
            D = jnp.where(take, D, Dq)
            IE = jnp.where(take, IE, IEq)
            R = jnp.where(take, R, Rq)

    er = jnp.exp(R - gamma)
    Lmat = (sub >= lane).astype(jnp.float32)
    colcum = jnp.dot(Lmat, er, preferred_element_type=jnp.float32,
                     precision=jax.lax.Precision.HIGHEST)
    tot = colcum[127:128, :]
    Umat = (sub < lane).astype(jnp.float32)
    off = jnp.dot(tot, Umat, preferred_element_type=jnp.float32,
                  precision=jax.lax.Precision.HIGHEST)
    S = colcum + off

    Es = (IE.astype(jnp.int32) & 1).astype(jnp.float32)
    contrib = Es * ((R - gamma) - jnp.log(S))
    n_ev = jnp.sum(Es)
    loss = -jnp.sum(contrib) / jnp.maximum(n_ev, 1.0)
    out_ref[...] = jnp.full((1, 1), loss, jnp.float32)


def _cox_sorted(d_cm, r_cm, e_cm, *, interpret=False):
    return pl.pallas_call(
        _body,
        out_shape=jax.ShapeDtypeStruct((1, 1), jnp.float32),
        interpret=interpret,
    )(d_cm, r_cm, e_cm)


def kernel(risk_scores, targets, *, interpret=False):
    r = risk_scores
    if r.ndim > 1:
        r = jnp.squeeze(r, axis=1)
    d = targets[:, 0]
    e = targets[:, 1]
    d_cm = d.reshape(128, 128).T
    r_cm = r.reshape(128, 128).T
    e_cm = e.reshape(128, 128).T
    out = _cox_sorted(d_cm, r_cm, e_cm, interpret=interpret)
    return out[0, 0]


# speedup vs baseline: 1.0554x; 1.0554x over previous
import jax
import jax.numpy as jnp
from jax.experimental import pallas as pl
from jax.experimental.pallas import tpu as pltpu


def _body(d_ref, r_ref, e_ref, out_ref):
    D = d_ref[...]
    R = r_ref[...]
    E = e_ref[...]
    sub = jax.lax.broadcasted_iota(jnp.int32, (128, 1), 0)
    lane = jax.lax.broadcasted_iota(jnp.int32, (1, 128), 1)
    I = lane * 128 + sub
    IE = I.astype(jnp.float32) * 2.0 + E
    gamma = jnp.max(R)
    n_ev = jnp.sum(E)
    er_sum_term = jnp.sum(E * (R - gamma))
    ER = jnp.exp(R - gamma)

    bits = [((I >> n) & 1) for n in range(14)]
    zero_bits = jnp.zeros((128, 128), jnp.int32)
    sub_bits = [(sub & (1 << n)) != 0 for n in range(7)]

    def pmat(t):
        return ((sub ^ t) == lane).astype(jnp.float32)

    def lane_partner(X, t, exact):
        prec = jax.lax.Precision.HIGHEST if exact else None
        return jnp.dot(X, pmat(t), preferred_element_type=jnp.float32,
                       precision=prec)

    def sub_partner(X, s):
        up = pltpu.roll(X, s, 0)
        dn = pltpu.roll(X, 128 - s, 0)
        n = s.bit_length() - 1
        return jnp.where(sub_bits[n], up, dn)

    for m in range(1, 15):
        bk = bits[m] if m < 14 else zero_bits
        for j_exp in range(m - 1, -1, -1):
            bj = bits[j_exp]
            keep = bk == bj
            if j_exp <= 6:
                s = 1 << j_exp
                Dq = sub_partner(D, s)
                IEq = sub_partner(IE, s)
                ERq = sub_partner(ER, s)
            else:
                t = 1 << (j_exp - 7)
                Dq = lane_partner(D, t, True)
                IEq = lane_partner(IE, t, True)
                ERq = lane_partner(ER, t, False)
            pre = (D > Dq) | ((D == Dq) & (IE < IEq))
            take = pre == keep
            D = jnp.where(take, D, Dq)
            IE = jnp.where(take, IE, IEq)
            ER = jnp.where(take, ER, ERq)

    Lmat = (sub >= lane).astype(jnp.float32)
    colcum = jnp.dot(Lmat, ER, preferred_element_type=jnp.float32,
                     precision=jax.lax.Precision.HIGHEST)
    tot = colcum[127:128, :]
    Umat = (sub < lane).astype(jnp.float32)
    off = jnp.dot(tot, Umat, preferred_element_type=jnp.float32,
                  precision=jax.lax.Precision.HIGHEST)
    S = colcum + off

    Es = (IE.astype(jnp.int32) & 1).astype(jnp.float32)
    log_term = jnp.sum(Es * jnp.log(S))
    loss = -(er_sum_term - log_term) / jnp.maximum(n_ev, 1.0)
    out_ref[...] = jnp.full((1, 1), loss, jnp.float32)


def _cox_sorted(d_cm, r_cm, e_cm, *, interpret=False):
    return pl.pallas_call(
        _body,
        out_shape=jax.ShapeDtypeStruct((1, 1), jnp.float32),
        interpret=interpret,
    )(d_cm, r_cm, e_cm)


def kernel(risk_scores, targets, *, interpret=False):
    r = risk_scores
    if r.ndim > 1:
        r = jnp.squeeze(r, axis=1)
    d = targets[:, 0]
    e = targets[:, 1]
    d_cm = d.reshape(128, 128).T
    r_cm = r.reshape(128, 128).T
    e_cm = e.reshape(128, 128).T
    out = _cox_sorted(d_cm, r_cm, e_cm, interpret=interpret)
    return out[0, 0]


# speedup vs baseline: 1.2851x; 1.2176x over previous
import jax
import jax.numpy as jnp
from jax.experimental import pallas as pl
from jax.experimental.pallas import tpu as pltpu


def _body(d_ref, r_ref, e_ref, out_ref):
    D = d_ref[...]
    R = r_ref[...]
    E = e_ref[...]
    sub = jax.lax.broadcasted_iota(jnp.int32, (128, 1), 0)
    lane = jax.lax.broadcasted_iota(jnp.int32, (1, 128), 1)
    I = lane * 128 + sub
    IE = I.astype(jnp.float32) * 2.0 + E
    gamma = jnp.max(R)
    n_ev = jnp.sum(E)
    er_sum_term = jnp.sum(E * (R - gamma))
    ER = jnp.exp(R - gamma)

    bits = [((I >> n) & 1) for n in range(14)]
    zero_bits = jnp.zeros((128, 128), jnp.int32)
    sub_bits = [(sub & (1 << n)) != 0 for n in range(7)]
    lane_bits = [(lane & (1 << n)) != 0 for n in range(7)]

    def xor_partner(X, s, axis, bitlist):
        up = pltpu.roll(X, s, axis)
        dn = pltpu.roll(X, 128 - s, axis)
        n = s.bit_length() - 1
        return jnp.where(bitlist[n], up, dn)

    for m in range(1, 15):
        bk = bits[m] if m < 14 else zero_bits
        for j_exp in range(m - 1, -1, -1):
            bj = bits[j_exp]
            keep = bk == bj
            if j_exp <= 6:
                s = 1 << j_exp
                Dq = xor_partner(D, s, 0, sub_bits)
                IEq = xor_partner(IE, s, 0, sub_bits)
                ERq = xor_partner(ER, s, 0, sub_bits)
            else:
                s = 1 << (j_exp - 7)
                Dq = xor_partner(D, s, 1, lane_bits)
                IEq = xor_partner(IE, s, 1, lane_bits)
                ERq = xor_partner(ER, s, 1, lane_bits)
            pre = (D > Dq) | ((D == Dq) & (IE < IEq))
            take = pre == keep
            D = jnp.where(take, D, Dq)
            IE = jnp.where(take, IE, IEq)
            ER = jnp.where(take, ER, ERq)

    Lmat = (sub >= lane).astype(jnp.float32)
    colcum = jnp.dot(Lmat, ER, preferred_element_type=jnp.float32,
                     precision=jax.lax.Precision.HIGHEST)
    tot = colcum[127:128, :]
    Umat = (sub < lane).astype(jnp.float32)
    off = jnp.dot(tot, Umat, preferred_element_type=jnp.float32,
                  precision=jax.lax.Precision.HIGHEST)
    S = colcum + off

    Es = (IE.astype(jnp.int32) & 1).astype(jnp.float32)
    log_term = jnp.sum(Es * jnp.log(S))
    loss = -(er_sum_term - log_term) / jnp.maximum(n_ev, 1.0)
    out_ref[...] = jnp.full((1, 1), loss, jnp.float32)


def _cox_sorted(d_cm, r_cm, e_cm, *, interpret=False):
    return pl.pallas_call(
        _body,
        out_shape=jax.ShapeDtypeStruct((1, 1), jnp.float32),
        interpret=interpret,
    )(d_cm, r_cm, e_cm)


def kernel(risk_scores, targets, *, interpret=False):
    r = risk_scores
    if r.ndim > 1:
        r = jnp.squeeze(r, axis=1)
    d = targets[:, 0]
    e = targets[:, 1]
    d_cm = d.reshape(128, 128).T
    r_cm = r.reshape(128, 128).T
    e_cm = e.reshape(128, 128).T
    out = _cox_sorted(d_cm, r_cm, e_cm, interpret=interpret)
    return out[0, 0]
